# Initial kernel scaffold; baseline (speedup 1.0000x reference)
import jax
import jax.numpy as jnp
from jax.experimental import pallas as pl


def kernel(x, edge_index, W1, b1, W2, b2):
    raise NotImplementedError("write your pallas kernel here")


# speedup vs baseline: 19.1352x; 19.1352x over previous
import functools

import jax
import jax.numpy as jnp
from jax import lax
from jax.experimental import pallas as pl
from jax.experimental.pallas import tpu as pltpu
from jax.experimental.pallas import tpu_sc as plsc

N = 10000
NP = 10240
E = 320000
F = 128
H = 64
CP = 16
CHUNK = 128
NCHUNKS = E // CHUNK
NC = 2
NS = 16
RPT = NP // NS

_mesh = plsc.VectorSubcoreMesh(core_axis_name="c", subcore_axis_name="s")


def _deg_body(dst_hbm, zeros_hbm, ones_hbm, out_hbm, acc, idx_v, ones_v):
    c = lax.axis_index("c")
    s = lax.axis_index("s")
    pltpu.sync_copy(zeros_hbm.at[pl.ds(s * RPT, RPT)], acc.at[pl.ds(s * RPT, RPT)])
    pltpu.sync_copy(ones_hbm, ones_v)
    plsc.subcore_barrier()
    half = NCHUNKS // NC
    nfull = half // NS
    n_iter = nfull + jnp.where(s < half - nfull * NS, 1, 0)

    def it(i, carry):
        g = c * half + s + i * NS
        off = g * CHUNK
        pltpu.sync_copy(dst_hbm.at[pl.ds(off, CHUNK)], idx_v.at[0])
        pltpu.sync_copy(ones_v, acc.at[idx_v.at[0]], add=True)
        return carry

    lax.fori_loop(0, n_iter, it, 0)
    plsc.subcore_barrier()
    pltpu.sync_copy(acc.at[pl.ds(s * RPT, RPT)], out_hbm.at[c].at[pl.ds(s * RPT, RPT)])


def _agg_body(d, h_hbm, src_hbm, dst_hbm, zeros_hbm, out_hbm, acc, src_v, dst_v, rows_v):
    c = lax.axis_index("c")
    s = lax.axis_index("s")
    pltpu.sync_copy(zeros_hbm.at[pl.ds(s * RPT, RPT)], acc.at[pl.ds(s * RPT, RPT)])
    plsc.subcore_barrier()
    half = NCHUNKS // NC
    nfull = half // NS
    n_iter = nfull + jnp.where(s < half - nfull * NS, 1, 0)

    def it(i, carry):
        g = c * half + s + i * NS
        off = g * CHUNK
        pltpu.sync_copy(src_hbm.at[pl.ds(off, CHUNK)], src_v)
        pltpu.sync_copy(h_hbm.at[src_v], rows_v)
        pltpu.sync_copy(dst_hbm.at[pl.ds(off, CHUNK)], dst_v.at[0])
        pltpu.sync_copy(rows_v, acc.at[dst_v.at[0]], add=True)
        return carry

    lax.fori_loop(0, n_iter, it, 0)
    plsc.subcore_barrier()
    pltpu.sync_copy(acc.at[pl.ds(s * RPT, RPT)], out_hbm.at[c].at[pl.ds(s * RPT, RPT)])


def _sc_degree(dst32, zeros1, ones1):
    return pl.kernel(
        _deg_body,
        out_type=jax.ShapeDtypeStruct((NC, NP, CP), jnp.float32),
        mesh=_mesh,
        scratch_types=[
            pltpu.VMEM_SHARED((NP, CP), jnp.float32),
            pltpu.VMEM((1, CHUNK), jnp.int32),
            pltpu.VMEM((CHUNK, CP), jnp.float32),
        ],
        compiler_params=pltpu.CompilerParams(use_tc_tiling_on_sc=False),
    )(dst32, zeros1, ones1)


def _sc_aggregate(d, h, src32, dst32, zerosd):
    body = functools.partial(_agg_body, d)
    return pl.kernel(
        body,
        out_type=jax.ShapeDtypeStruct((NC, NP, d), jnp.float32),
        mesh=_mesh,
        scratch_types=[
            pltpu.VMEM_SHARED((NP, d), jnp.float32),
            pltpu.VMEM((CHUNK,), jnp.int32),
            pltpu.VMEM((1, CHUNK), jnp.int32),
            pltpu.VMEM((CHUNK, d), jnp.float32),
        ],
        compiler_params=pltpu.CompilerParams(use_tc_tiling_on_sc=False),
    )(h, src32, dst32, zerosd)


def _tc_pre_body(x_ref, w1_ref, da_ref, db_ref, h1s_ref, dis_ref):
    deg = da_ref[...] + db_ref[...] + 1.0
    dis = lax.rsqrt(deg)
    dis_ref[...] = dis
    h1s_ref[...] = jnp.dot(x_ref[...], w1_ref[...], preferred_element_type=jnp.float32) * dis


def _tc_mid_body(pa_ref, pb_ref, h1s_ref, dis_ref, b1_ref, w2_ref, h2s_ref):
    dis = dis_ref[...]
    y1 = pa_ref[...] + pb_ref[...] + h1s_ref[...]
    r = jnp.maximum(dis * y1 + b1_ref[...], 0.0)
    h2s_ref[...] = jnp.dot(r, w2_ref[...], preferred_element_type=jnp.float32) * dis


def _tc_post_body(pa_ref, pb_ref, h2s_ref, dis_ref, b2_ref, out_ref):
    y2 = pa_ref[...] + pb_ref[...] + h2s_ref[...]
    out_ref[...] = dis_ref[...] * y2 + b2_ref[...]


def kernel(x, edge_index, W1, b1, W2, b2):
    src32 = edge_index[0].astype(jnp.int32)
    dst32 = edge_index[1].astype(jnp.int32)
    zeros1 = jnp.zeros((NP, CP), jnp.float32)
    ones1 = jnp.ones((CHUNK, CP), jnp.float32)
    zeros64 = jnp.zeros((NP, H), jnp.float32)
    zeros16 = jnp.zeros((NP, CP), jnp.float32)
    W2p = jnp.concatenate([W2, jnp.zeros((H, CP - W2.shape[1]), jnp.float32)], axis=1)
    b2p = jnp.concatenate([b2, jnp.zeros((CP - b2.shape[0],), jnp.float32)]).reshape(1, CP)
    b1r = b1.reshape(1, H)

    degp = _sc_degree(dst32, zeros1, ones1)

    h1s, dis = pl.pallas_call(
        _tc_pre_body,
        out_shape=[
            jax.ShapeDtypeStruct((N, H), jnp.float32),
            jax.ShapeDtypeStruct((N, 1), jnp.float32),
        ],
    )(x, W1, degp[0, :N, 0:1], degp[1, :N, 0:1])

    agg1 = _sc_aggregate(H, h1s, src32, dst32, zeros64)

    h2s = pl.pallas_call(
        _tc_mid_body,
        out_shape=jax.ShapeDtypeStruct((N, CP), jnp.float32),
    )(agg1[0, :N], agg1[1, :N], h1s, dis, b1r, W2p)

    agg2 = _sc_aggregate(CP, h2s, src32, dst32, zeros16)

    out16 = pl.pallas_call(
        _tc_post_body,
        out_shape=jax.ShapeDtypeStruct((N, CP), jnp.float32),
    )(agg2[0, :N], agg2[1, :N], h2s, dis, b2p)

    return out16[:, : b2.shape[0]]


# speedup vs baseline: 19.8026x; 1.0349x over previous
import functools

import jax
import jax.numpy as jnp
from jax import lax
from jax.experimental import pallas as pl
from jax.experimental.pallas import tpu as pltpu
from jax.experimental.pallas import tpu_sc as plsc

N = 10000
NP = 10240
E = 320000
F = 128
H = 64
CP = 16
CHUNK = 128
NC = 2
NS = 16
NW = NC * NS
NBUF = 8
NGRP = 10
CPT = NBUF * NGRP
EP = NW * CPT * CHUNK
RPT = NP // NS

_mesh = plsc.VectorSubcoreMesh(core_axis_name="c", subcore_axis_name="s")


def _deg_body(dst_hbm, zeros_hbm, ones_hbm, out_hbm, acc, dstb, ones_v, ssem):
    c = lax.axis_index("c")
    s = lax.axis_index("s")
    w = c * NS + s
    pltpu.sync_copy(zeros_hbm.at[pl.ds(s * RPT, RPT)], acc.at[pl.ds(s * RPT, RPT)])
    pltpu.sync_copy(ones_hbm, ones_v)
    pltpu.sync_copy(dst_hbm.at[pl.ds(w * CPT, CPT)], dstb)
    plsc.subcore_barrier()

    def grp(g, carry):
        for b in range(NBUF):
            pltpu.async_copy(ones_v, acc.at[dstb.at[g * NBUF + b]], ssem.at[b], add=True)
        for b in range(NBUF):
            pltpu.make_async_copy(ones_v, acc.at[dstb.at[g * NBUF + b]], ssem.at[b]).wait()
        return carry

    lax.fori_loop(0, NGRP, grp, 0)
    plsc.subcore_barrier()
    pltpu.sync_copy(acc.at[pl.ds(s * RPT, RPT)], out_hbm.at[c].at[pl.ds(s * RPT, RPT)])


def _agg_body(d, h_hbm, src_hbm, dst_hbm, zeros_hbm, out_hbm,
              acc, srcb, dstb, rows, gsem, ssem):
    c = lax.axis_index("c")
    s = lax.axis_index("s")
    w = c * NS + s
    pltpu.sync_copy(zeros_hbm.at[pl.ds(s * RPT, RPT)], acc.at[pl.ds(s * RPT, RPT)])
    pltpu.sync_copy(src_hbm.at[pl.ds(w * CPT, CPT)], srcb)
    pltpu.sync_copy(dst_hbm.at[pl.ds(w * CPT, CPT)], dstb)
    plsc.subcore_barrier()

    for b in range(NBUF):
        pltpu.async_copy(h_hbm.at[srcb.at[b]], rows.at[b], gsem.at[b])

    def grp(g, carry):
        for b in range(NBUF):
            j = g * NBUF + b
            pltpu.make_async_copy(h_hbm.at[srcb.at[j]], rows.at[b], gsem.at[b]).wait()
            pltpu.async_copy(rows.at[b], acc.at[dstb.at[j]], ssem.at[b], add=True)
        for b in range(NBUF):
            j = g * NBUF + b
            pltpu.make_async_copy(rows.at[b], acc.at[dstb.at[j]], ssem.at[b]).wait()
            jn = jnp.minimum(j + NBUF, CPT - 1)
            pltpu.async_copy(h_hbm.at[srcb.at[jn]], rows.at[b], gsem.at[b])
        return carry

    lax.fori_loop(0, NGRP, grp, 0)
    for b in range(NBUF):
        pltpu.make_async_copy(h_hbm.at[srcb.at[CPT - 1]], rows.at[b], gsem.at[b]).wait()
    plsc.subcore_barrier()
    pltpu.sync_copy(acc.at[pl.ds(s * RPT, RPT)], out_hbm.at[c].at[pl.ds(s * RPT, RPT)])


def _sc_degree(dst2d, zeros1, ones1):
    return pl.kernel(
        _deg_body,
        out_type=jax.ShapeDtypeStruct((NC, NP, CP), jnp.float32),
        mesh=_mesh,
        scratch_types=[
            pltpu.VMEM_SHARED((NP, CP), jnp.float32),
            pltpu.VMEM((CPT, CHUNK), jnp.int32),
            pltpu.VMEM((CHUNK, CP), jnp.float32),
            pltpu.SemaphoreType.DMA((NBUF,)),
        ],
        compiler_params=pltpu.CompilerParams(use_tc_tiling_on_sc=False),
    )(dst2d, zeros1, ones1)


def _sc_aggregate(d, h, src2d, dst2d, zerosd):
    body = functools.partial(_agg_body, d)
    return pl.kernel(
        body,
        out_type=jax.ShapeDtypeStruct((NC, NP, d), jnp.float32),
        mesh=_mesh,
        scratch_types=[
            pltpu.VMEM_SHARED((NP, d), jnp.float32),
            pltpu.VMEM((CPT, CHUNK), jnp.int32),
            pltpu.VMEM((CPT, CHUNK), jnp.int32),
            pltpu.VMEM((NBUF, CHUNK, d), jnp.float32),
            pltpu.SemaphoreType.DMA((NBUF,)),
            pltpu.SemaphoreType.DMA((NBUF,)),
        ],
        compiler_params=pltpu.CompilerParams(use_tc_tiling_on_sc=False),
    )(h, src2d, dst2d, zerosd)


def _tc_pre_body(x_ref, w1_ref, da_ref, db_ref, h1s_ref, dis_ref):
    deg = da_ref[...] + db_ref[...] + 1.0
    dis = lax.rsqrt(deg)
    dis_ref[...] = dis
    h1s_ref[...] = jnp.dot(x_ref[...], w1_ref[...], preferred_element_type=jnp.float32) * dis


def _tc_mid_body(pa_ref, pb_ref, h1s_ref, dis_ref, b1_ref, w2_ref, h2s_ref):
    dis = dis_ref[...]
    y1 = pa_ref[...] + pb_ref[...] + h1s_ref[...]
    r = jnp.maximum(dis * y1 + b1_ref[...], 0.0)
    h2s_ref[...] = jnp.dot(r, w2_ref[...], preferred_element_type=jnp.float32) * dis


def _tc_post_body(pa_ref, pb_ref, h2s_ref, dis_ref, b2_ref, out_ref):
    y2 = pa_ref[...] + pb_ref[...] + h2s_ref[...]
    out_ref[...] = dis_ref[...] * y2 + b2_ref[...]


def kernel(x, edge_index, W1, b1, W2, b2):
    src32 = edge_index[0].astype(jnp.int32)
    dst32 = edge_index[1].astype(jnp.int32)
    pad = EP - E
    src2d = jnp.concatenate([src32, jnp.zeros((pad,), jnp.int32)]).reshape(EP // CHUNK, CHUNK)
    dst2d = jnp.concatenate([dst32, jnp.full((pad,), N, jnp.int32)]).reshape(EP // CHUNK, CHUNK)
    zeros1 = jnp.zeros((NP, CP), jnp.float32)
    ones1 = jnp.ones((CHUNK, CP), jnp.float32)
    zeros64 = jnp.zeros((NP, H), jnp.float32)
    zeros16 = jnp.zeros((NP, CP), jnp.float32)
    W2p = jnp.concatenate([W2, jnp.zeros((H, CP - W2.shape[1]), jnp.float32)], axis=1)
    b2p = jnp.concatenate([b2, jnp.zeros((CP - b2.shape[0],), jnp.float32)]).reshape(1, CP)
    b1r = b1.reshape(1, H)

    degp = _sc_degree(dst2d, zeros1, ones1)

    h1s, dis = pl.pallas_call(
        _tc_pre_body,
        out_shape=[
            jax.ShapeDtypeStruct((N, H), jnp.float32),
            jax.ShapeDtypeStruct((N, 1), jnp.float32),
        ],
    )(x, W1, degp[0, :N, 0:1], degp[1, :N, 0:1])

    agg1 = _sc_aggregate(H, h1s, src2d, dst2d, zeros64)

    h2s = pl.pallas_call(
        _tc_mid_body,
        out_shape=jax.ShapeDtypeStruct((N, CP), jnp.float32),
    )(agg1[0, :N], agg1[1, :N], h1s, dis, b1r, W2p)

    agg2 = _sc_aggregate(CP, h2s, src2d, dst2d, zeros16)

    out16 = pl.pallas_call(
        _tc_post_body,
        out_shape=jax.ShapeDtypeStruct((N, CP), jnp.float32),
    )(agg2[0, :N], agg2[1, :N], h2s, dis, b2p)

    return out16[:, : b2.shape[0]]


# speedup vs baseline: 20.0798x; 1.0140x over previous
import functools

import jax
import jax.numpy as jnp
from jax import lax
from jax.experimental import pallas as pl
from jax.experimental.pallas import tpu as pltpu
from jax.experimental.pallas import tpu_sc as plsc

N = 10000
NP = 10240
E = 320000
F = 128
H = 64
CP = 16
CHUNK = 128
NC = 2
NS = 16
NW = NC * NS
NBUF = 8
NGRP = 10
CPT = NBUF * NGRP
EP = NW * CPT * CHUNK
RPT = NP // NS

_mesh = plsc.VectorSubcoreMesh(core_axis_name="c", subcore_axis_name="s")


def _deg_body(dst_hbm, zeros_hbm, ones_hbm, out_hbm, acc, dstb, ones_v, ssem):
    c = lax.axis_index("c")
    s = lax.axis_index("s")
    w = c * NS + s
    pltpu.sync_copy(zeros_hbm.at[pl.ds(s * RPT, RPT)], acc.at[pl.ds(s * RPT, RPT)])
    pltpu.sync_copy(ones_hbm, ones_v)
    pltpu.sync_copy(dst_hbm.at[pl.ds(w * CPT, CPT)], dstb)
    plsc.subcore_barrier()

    def grp(g, carry):
        for b in range(NBUF):
            pltpu.async_copy(ones_v, acc.at[dstb.at[g * NBUF + b]], ssem.at[b], add=True)
        for b in range(NBUF):
            pltpu.make_async_copy(ones_v, acc.at[dstb.at[g * NBUF + b]], ssem.at[b]).wait()
        return carry

    lax.fori_loop(0, NGRP, grp, 0)
    plsc.subcore_barrier()
    pltpu.sync_copy(acc.at[pl.ds(s * RPT, RPT)], out_hbm.at[c].at[pl.ds(s * RPT, RPT)])


def _agg_body(d, h_hbm, src_hbm, dst_hbm, zeros_hbm, out_hbm,
              acc, srcb, dstb, rows, gsem, ssem):
    c = lax.axis_index("c")
    s = lax.axis_index("s")
    w = c * NS + s
    pltpu.sync_copy(zeros_hbm.at[pl.ds(s * RPT, RPT)], acc.at[pl.ds(s * RPT, RPT)])
    pltpu.sync_copy(src_hbm.at[pl.ds(w * CPT, CPT)], srcb)
    pltpu.sync_copy(dst_hbm.at[pl.ds(w * CPT, CPT)], dstb)
    plsc.subcore_barrier()

    for b in range(NBUF):
        pltpu.async_copy(h_hbm.at[srcb.at[b]], rows.at[b], gsem.at[b])

    def grp(g, carry):
        for b in range(NBUF):
            j = g * NBUF + b
            pltpu.make_async_copy(h_hbm.at[srcb.at[j]], rows.at[b], gsem.at[b]).wait()
            pltpu.async_copy(rows.at[b], acc.at[dstb.at[j]], ssem.at[b], add=True)
        for b in range(NBUF):
            j = g * NBUF + b
            pltpu.make_async_copy(rows.at[b], acc.at[dstb.at[j]], ssem.at[b]).wait()
            jn = jnp.minimum(j + NBUF, CPT - 1)
            pltpu.async_copy(h_hbm.at[srcb.at[jn]], rows.at[b], gsem.at[b])
        return carry

    lax.fori_loop(0, NGRP, grp, 0)
    for b in range(NBUF):
        pltpu.make_async_copy(h_hbm.at[srcb.at[CPT - 1]], rows.at[b], gsem.at[b]).wait()
    plsc.subcore_barrier()
    pltpu.sync_copy(acc.at[pl.ds(s * RPT, RPT)], out_hbm.at[c].at[pl.ds(s * RPT, RPT)])


def _sc_degree(dst2d, zeros1, ones1):
    return pl.kernel(
        _deg_body,
        out_type=jax.ShapeDtypeStruct((NC, NP, CP), jnp.float32),
        mesh=_mesh,
        scratch_types=[
            pltpu.VMEM_SHARED((NP, CP), jnp.float32),
            pltpu.VMEM((CPT, CHUNK), jnp.int32),
            pltpu.VMEM((CHUNK, CP), jnp.float32),
            pltpu.SemaphoreType.DMA((NBUF,)),
        ],
        compiler_params=pltpu.CompilerParams(use_tc_tiling_on_sc=False),
    )(dst2d, zeros1, ones1)


def _sc_aggregate(d, h, src2d, dst2d, zerosd):
    body = functools.partial(_agg_body, d)
    return pl.kernel(
        body,
        out_type=jax.ShapeDtypeStruct((NC, NP, d), jnp.float32),
        mesh=_mesh,
        scratch_types=[
            pltpu.VMEM_SHARED((NP, d), jnp.float32),
            pltpu.VMEM((CPT, CHUNK), jnp.int32),
            pltpu.VMEM((CPT, CHUNK), jnp.int32),
            pltpu.VMEM((NBUF, CHUNK, d), jnp.float32),
            pltpu.SemaphoreType.DMA((NBUF,)),
            pltpu.SemaphoreType.DMA((NBUF,)),
        ],
        compiler_params=pltpu.CompilerParams(use_tc_tiling_on_sc=False),
    )(h, src2d, dst2d, zerosd)


def _tc_pre_body(x_ref, w1_ref, da_ref, db_ref, h1s_ref, dis_ref):
    deg = da_ref[...] + db_ref[...] + 1.0
    dis = lax.rsqrt(deg)
    dis_ref[...] = dis
    h1s_ref[...] = jnp.dot(x_ref[...], w1_ref[...], preferred_element_type=jnp.float32) * dis


def _tc_mid_body(pa_ref, pb_ref, h1s_ref, dis_ref, b1_ref, w2_ref, h2s_ref):
    dis = dis_ref[...]
    y1 = pa_ref[...] + pb_ref[...] + h1s_ref[...]
    r = jnp.maximum(dis * y1 + b1_ref[...], 0.0)
    h2s_ref[...] = jnp.dot(r, w2_ref[...], preferred_element_type=jnp.float32) * dis


def _tc_post_body(pa_ref, pb_ref, h2s_ref, dis_ref, b2_ref, out_ref):
    y2 = pa_ref[...] + pb_ref[...] + h2s_ref[...]
    out_ref[...] = dis_ref[...] * y2 + b2_ref[...]


def kernel(x, edge_index, W1, b1, W2, b2):
    src32 = edge_index[0].astype(jnp.int32)
    dst32 = edge_index[1].astype(jnp.int32)
    pad = EP - E
    src2d = jnp.concatenate([src32, jnp.zeros((pad,), jnp.int32)]).reshape(EP // CHUNK, CHUNK)
    pad_dst = N + jnp.arange(pad, dtype=jnp.int32) % (NP - N)
    dst2d = jnp.concatenate([dst32, pad_dst]).reshape(EP // CHUNK, CHUNK)
    zeros1 = jnp.zeros((NP, CP), jnp.float32)
    ones1 = jnp.ones((CHUNK, CP), jnp.float32)
    zeros64 = jnp.zeros((NP, H), jnp.float32)
    zeros16 = jnp.zeros((NP, CP), jnp.float32)
    W2p = jnp.concatenate([W2, jnp.zeros((H, CP - W2.shape[1]), jnp.float32)], axis=1)
    b2p = jnp.concatenate([b2, jnp.zeros((CP - b2.shape[0],), jnp.float32)]).reshape(1, CP)
    b1r = b1.reshape(1, H)

    degp = _sc_degree(dst2d, zeros1, ones1)

    h1s, dis = pl.pallas_call(
        _tc_pre_body,
        out_shape=[
            jax.ShapeDtypeStruct((N, H), jnp.float32),
            jax.ShapeDtypeStruct((N, 1), jnp.float32),
        ],
    )(x, W1, degp[0, :N, 0:1], degp[1, :N, 0:1])

    agg1 = _sc_aggregate(H, h1s, src2d, dst2d, zeros64)

    h2s = pl.pallas_call(
        _tc_mid_body,
        out_shape=jax.ShapeDtypeStruct((N, CP), jnp.float32),
    )(agg1[0, :N], agg1[1, :N], h1s, dis, b1r, W2p)

    agg2 = _sc_aggregate(CP, h2s, src2d, dst2d, zeros16)

    out16 = pl.pallas_call(
        _tc_post_body,
        out_shape=jax.ShapeDtypeStruct((N, CP), jnp.float32),
    )(agg2[0, :N], agg2[1, :N], h2s, dis, b2p)

    return out16[:, : b2.shape[0]]


# speedup vs baseline: 34.2207x; 1.7042x over previous
import functools

import jax
import jax.numpy as jnp
from jax import lax
from jax.experimental import pallas as pl
from jax.experimental.pallas import tpu as pltpu
from jax.experimental.pallas import tpu_sc as plsc

N = 10000
NP = 10240
E = 320000
F = 128
H = 64
HS = 32
CP = 16
CHUNK = 128
NC = 2
NS = 16
NW = NC * NS
NBUF = 8
NGRP = 10
CPT = NBUF * NGRP
EP = NW * CPT * CHUNK
RPT = NP // NS

_mesh = plsc.VectorSubcoreMesh(core_axis_name="c", subcore_axis_name="s")


def _deg_body(dst_hbm, zeros_hbm, ones_hbm, out_hbm, acc, dstb, ones_v, ssem):
    c = lax.axis_index("c")
    s = lax.axis_index("s")
    w = c * NS + s
    pltpu.sync_copy(zeros_hbm.at[pl.ds(s * RPT, RPT)], acc.at[pl.ds(s * RPT, RPT)])
    pltpu.sync_copy(ones_hbm, ones_v)
    pltpu.sync_copy(dst_hbm.at[pl.ds(w * CPT, CPT)], dstb)
    plsc.subcore_barrier()

    def grp(g, carry):
        for b in range(NBUF):
            pltpu.async_copy(ones_v, acc.at[dstb.at[g * NBUF + b]], ssem.at[b], add=True)
        for b in range(NBUF):
            pltpu.make_async_copy(ones_v, acc.at[dstb.at[g * NBUF + b]], ssem.at[b]).wait()
        return carry

    lax.fori_loop(0, NGRP, grp, 0)
    plsc.subcore_barrier()
    pltpu.sync_copy(acc.at[pl.ds(s * RPT, RPT)], out_hbm.at[c].at[pl.ds(s * RPT, RPT)])


def _agg_body(nsplit, d, h_hbm, src_hbm, dst_hbm, zeros_hbm, out_hbm,
              acc, hstage, srcb, dstb, rows, gsem, ssem):
    c = lax.axis_index("c")
    s = lax.axis_index("s")
    w = c * NS + s
    pltpu.sync_copy(src_hbm.at[pl.ds(w * CPT, CPT)], srcb)
    pltpu.sync_copy(dst_hbm.at[pl.ds(w * CPT, CPT)], dstb)

    for kp in range(nsplit):
        pltpu.sync_copy(zeros_hbm.at[pl.ds(s * RPT, RPT)], acc.at[pl.ds(s * RPT, RPT)])
        pltpu.sync_copy(h_hbm.at[kp].at[pl.ds(s * RPT, RPT)], hstage.at[pl.ds(s * RPT, RPT)])
        plsc.subcore_barrier()

        for b in range(NBUF):
            pltpu.async_copy(hstage.at[srcb.at[b]], rows.at[b], gsem.at[b])

        def grp(g, carry):
            for b in range(NBUF):
                j = g * NBUF + b
                pltpu.make_async_copy(hstage.at[srcb.at[j]], rows.at[b], gsem.at[b]).wait()
                pltpu.async_copy(rows.at[b], acc.at[dstb.at[j]], ssem.at[b], add=True)
            for b in range(NBUF):
                j = g * NBUF + b
                pltpu.make_async_copy(rows.at[b], acc.at[dstb.at[j]], ssem.at[b]).wait()
                jn = jnp.minimum(j + NBUF, CPT - 1)
                pltpu.async_copy(hstage.at[srcb.at[jn]], rows.at[b], gsem.at[b])
            return carry

        lax.fori_loop(0, NGRP, grp, 0)
        for b in range(NBUF):
            pltpu.make_async_copy(hstage.at[srcb.at[CPT - 1]], rows.at[b], gsem.at[b]).wait()
        plsc.subcore_barrier()
        pltpu.sync_copy(acc.at[pl.ds(s * RPT, RPT)],
                        out_hbm.at[c].at[kp].at[pl.ds(s * RPT, RPT)])


def _sc_degree(dst2d, zeros1, ones1):
    return pl.kernel(
        _deg_body,
        out_type=jax.ShapeDtypeStruct((NC, NP, CP), jnp.float32),
        mesh=_mesh,
        scratch_types=[
            pltpu.VMEM_SHARED((NP, CP), jnp.float32),
            pltpu.VMEM((CPT, CHUNK), jnp.int32),
            pltpu.VMEM((CHUNK, CP), jnp.float32),
            pltpu.SemaphoreType.DMA((NBUF,)),
        ],
        compiler_params=pltpu.CompilerParams(use_tc_tiling_on_sc=False),
    )(dst2d, zeros1, ones1)


def _sc_aggregate(nsplit, d, h, src2d, dst2d, zerosd):
    body = functools.partial(_agg_body, nsplit, d)
    return pl.kernel(
        body,
        out_type=jax.ShapeDtypeStruct((NC, nsplit, NP, d), jnp.float32),
        mesh=_mesh,
        scratch_types=[
            pltpu.VMEM_SHARED((NP, d), jnp.float32),
            pltpu.VMEM_SHARED((NP, d), jnp.float32),
            pltpu.VMEM((CPT, CHUNK), jnp.int32),
            pltpu.VMEM((CPT, CHUNK), jnp.int32),
            pltpu.VMEM((NBUF, CHUNK, d), jnp.float32),
            pltpu.SemaphoreType.DMA((NBUF,)),
            pltpu.SemaphoreType.DMA((NBUF,)),
        ],
        compiler_params=pltpu.CompilerParams(use_tc_tiling_on_sc=False),
    )(h, src2d, dst2d, zerosd)


def _tc_pre_body(x_ref, w1_ref, da_ref, db_ref, h1s_ref, dis_ref):
    deg = da_ref[...] + db_ref[...] + 1.0
    dis = lax.rsqrt(deg)
    dis_ref[...] = dis
    res = jnp.dot(x_ref[...], w1_ref[...], preferred_element_type=jnp.float32) * dis
    h1s_ref[0, pl.ds(0, N)] = res[:, :HS]
    h1s_ref[1, pl.ds(0, N)] = res[:, HS:]
    h1s_ref[0, pl.ds(N, NP - N)] = jnp.zeros((NP - N, HS), jnp.float32)
    h1s_ref[1, pl.ds(N, NP - N)] = jnp.zeros((NP - N, HS), jnp.float32)


def _tc_mid_body(pa_ref, pb_ref, h1s_ref, dis_ref, b1_ref, w2a_ref, w2b_ref, h2s_ref):
    dis = dis_ref[...]
    yl = pa_ref[0, pl.ds(0, N)] + pb_ref[0, pl.ds(0, N)] + h1s_ref[0, pl.ds(0, N)]
    yr = pa_ref[1, pl.ds(0, N)] + pb_ref[1, pl.ds(0, N)] + h1s_ref[1, pl.ds(0, N)]
    rl = jnp.maximum(dis * yl + b1_ref[:, :HS], 0.0)
    rr = jnp.maximum(dis * yr + b1_ref[:, HS:], 0.0)
    h2 = (jnp.dot(rl, w2a_ref[...], preferred_element_type=jnp.float32)
          + jnp.dot(rr, w2b_ref[...], preferred_element_type=jnp.float32))
    h2s_ref[pl.ds(0, N)] = h2 * dis
    h2s_ref[pl.ds(N, NP - N)] = jnp.zeros((NP - N, CP), jnp.float32)


def _tc_post_body(pa_ref, pb_ref, h2s_ref, dis_ref, b2_ref, out_ref):
    y2 = pa_ref[...] + pb_ref[...] + h2s_ref[...]
    out_ref[...] = dis_ref[...] * y2 + b2_ref[...]


def kernel(x, edge_index, W1, b1, W2, b2):
    src32 = edge_index[0].astype(jnp.int32)
    dst32 = edge_index[1].astype(jnp.int32)
    pad = EP - E
    src2d = jnp.concatenate([src32, jnp.zeros((pad,), jnp.int32)]).reshape(EP // CHUNK, CHUNK)
    pad_dst = N + jnp.arange(pad, dtype=jnp.int32) % (NP - N)
    dst2d = jnp.concatenate([dst32, pad_dst]).reshape(EP // CHUNK, CHUNK)
    zeros1 = jnp.zeros((NP, CP), jnp.float32)
    ones1 = jnp.ones((CHUNK, CP), jnp.float32)
    zeros32 = jnp.zeros((NP, HS), jnp.float32)
    zeros16 = jnp.zeros((NP, CP), jnp.float32)
    W2p = jnp.concatenate([W2, jnp.zeros((H, CP - W2.shape[1]), jnp.float32)], axis=1)
    b2p = jnp.concatenate([b2, jnp.zeros((CP - b2.shape[0],), jnp.float32)]).reshape(1, CP)
    b1r = b1.reshape(1, H)

    degp = _sc_degree(dst2d, zeros1, ones1)

    h1s, dis = pl.pallas_call(
        _tc_pre_body,
        out_shape=[
            jax.ShapeDtypeStruct((2, NP, HS), jnp.float32),
            jax.ShapeDtypeStruct((N, 1), jnp.float32),
        ],
    )(x, W1, degp[0, :N, 0:1], degp[1, :N, 0:1])

    agg1 = _sc_aggregate(2, HS, h1s, src2d, dst2d, zeros32)

    h2s = pl.pallas_call(
        _tc_mid_body,
        out_shape=jax.ShapeDtypeStruct((NP, CP), jnp.float32),
    )(agg1[0], agg1[1], h1s, dis, b1r, W2p[:HS], W2p[HS:])

    agg2 = _sc_aggregate(1, CP, h2s.reshape(1, NP, CP), src2d, dst2d, zeros16)

    out16 = pl.pallas_call(
        _tc_post_body,
        out_shape=jax.ShapeDtypeStruct((N, CP), jnp.float32),
    )(agg2[0, 0, :N], agg2[1, 0, :N], h2s[:N], dis, b2p)

    return out16[:, : b2.shape[0]]


# speedup vs baseline: 38.8525x; 1.1354x over previous
import functools

import jax
import jax.numpy as jnp
from jax import lax
from jax.experimental import pallas as pl
from jax.experimental.pallas import tpu as pltpu
from jax.experimental.pallas import tpu_sc as plsc

N = 10000
NP = 10240
E = 320000
F = 128
H = 64
HS = 32
CP = 16
CHUNK = 128
NC = 2
NS = 16
NW = NC * NS
NBUF = 8
NGRP = 10
CPT = NBUF * NGRP
EP = NW * CPT * CHUNK
RPT = NP // NS
BN = 1000

_mesh = plsc.VectorSubcoreMesh(core_axis_name="c", subcore_axis_name="s")


def _deg_body(dst_hbm, zeros_hbm, ones_hbm, out_hbm, acc, dstb, ones_v, ssem):
    c = lax.axis_index("c")
    s = lax.axis_index("s")
    w = c * NS + s
    pltpu.sync_copy(zeros_hbm.at[pl.ds(s * RPT, RPT)], acc.at[pl.ds(s * RPT, RPT)])
    pltpu.sync_copy(ones_hbm, ones_v)
    pltpu.sync_copy(dst_hbm.at[pl.ds(w * CPT, CPT)], dstb)
    plsc.subcore_barrier()

    def grp(g, carry):
        for b in range(NBUF):
            pltpu.async_copy(ones_v, acc.at[dstb.at[g * NBUF + b]], ssem.at[b], add=True)
        for b in range(NBUF):
            pltpu.make_async_copy(ones_v, acc.at[dstb.at[g * NBUF + b]], ssem.at[b]).wait()
        return carry

    lax.fori_loop(0, NGRP, grp, 0)
    plsc.subcore_barrier()
    pltpu.sync_copy(acc.at[pl.ds(s * RPT, RPT)], out_hbm.at[c].at[pl.ds(s * RPT, RPT)])


def _agg_body(nsplit, d, h_hbm, src_hbm, dst_hbm, zeros_hbm, out_hbm,
              acc, hstage, srcb, dstb, rows, gsem, ssem):
    c = lax.axis_index("c")
    s = lax.axis_index("s")
    w = c * NS + s
    pltpu.sync_copy(src_hbm.at[pl.ds(w * CPT, CPT)], srcb)
    pltpu.sync_copy(dst_hbm.at[pl.ds(w * CPT, CPT)], dstb)

    for kp in range(nsplit):
        @pl.when(c == 0)
        def _():
            pltpu.sync_copy(h_hbm.at[kp].at[pl.ds(s * RPT, RPT)], acc.at[pl.ds(s * RPT, RPT)])

        @pl.when(c != 0)
        def _():
            pltpu.sync_copy(zeros_hbm.at[pl.ds(s * RPT, RPT)], acc.at[pl.ds(s * RPT, RPT)])

        pltpu.sync_copy(h_hbm.at[kp].at[pl.ds(s * RPT, RPT)], hstage.at[pl.ds(s * RPT, RPT)])
        plsc.subcore_barrier()

        for b in range(NBUF):
            pltpu.async_copy(hstage.at[srcb.at[b]], rows.at[b], gsem.at[b])

        def grp(g, carry):
            for b in range(NBUF):
                j = g * NBUF + b
                pltpu.make_async_copy(hstage.at[srcb.at[j]], rows.at[b], gsem.at[b]).wait()
                pltpu.async_copy(rows.at[b], acc.at[dstb.at[j]], ssem.at[b], add=True)
            for b in range(NBUF):
                j = g * NBUF + b
                pltpu.make_async_copy(rows.at[b], acc.at[dstb.at[j]], ssem.at[b]).wait()
                pltpu.async_copy(hstage.at[srcb.at[j + NBUF]], rows.at[b], gsem.at[b])
            return carry

        lax.fori_loop(0, NGRP - 1, grp, 0)
        for b in range(NBUF):
            j = (NGRP - 1) * NBUF + b
            pltpu.make_async_copy(hstage.at[srcb.at[j]], rows.at[b], gsem.at[b]).wait()
            pltpu.async_copy(rows.at[b], acc.at[dstb.at[j]], ssem.at[b], add=True)
        for b in range(NBUF):
            j = (NGRP - 1) * NBUF + b
            pltpu.make_async_copy(rows.at[b], acc.at[dstb.at[j]], ssem.at[b]).wait()
        plsc.subcore_barrier()
        pltpu.sync_copy(acc.at[pl.ds(s * RPT, RPT)],
                        out_hbm.at[c].at[kp].at[pl.ds(s * RPT, RPT)])


def _sc_degree(dst2d, zeros1, ones1):
    return pl.kernel(
        _deg_body,
        out_type=jax.ShapeDtypeStruct((NC, NP, CP), jnp.float32),
        mesh=_mesh,
        scratch_types=[
            pltpu.VMEM_SHARED((NP, CP), jnp.float32),
            pltpu.VMEM((CPT, CHUNK), jnp.int32),
            pltpu.VMEM((CHUNK, CP), jnp.float32),
            pltpu.SemaphoreType.DMA((NBUF,)),
        ],
        compiler_params=pltpu.CompilerParams(use_tc_tiling_on_sc=False),
    )(dst2d, zeros1, ones1)


def _sc_aggregate(nsplit, d, h, src2d, dst2d, zerosd):
    body = functools.partial(_agg_body, nsplit, d)
    return pl.kernel(
        body,
        out_type=jax.ShapeDtypeStruct((NC, nsplit, NP, d), jnp.float32),
        mesh=_mesh,
        scratch_types=[
            pltpu.VMEM_SHARED((NP, d), jnp.float32),
            pltpu.VMEM_SHARED((NP, d), jnp.float32),
            pltpu.VMEM((CPT, CHUNK), jnp.int32),
            pltpu.VMEM((CPT, CHUNK), jnp.int32),
            pltpu.VMEM((NBUF, CHUNK, d), jnp.float32),
            pltpu.SemaphoreType.DMA((NBUF,)),
            pltpu.SemaphoreType.DMA((NBUF,)),
        ],
        compiler_params=pltpu.CompilerParams(use_tc_tiling_on_sc=False),
    )(h, src2d, dst2d, zerosd)


def _tc_pre_body(x_ref, w1_ref, degp_ref, h1s_ref, dis_ref):
    deg = degp_ref[0, :, 0:1] + degp_ref[1, :, 0:1] + 1.0
    dis = lax.rsqrt(deg)
    dis_ref[...] = dis
    res = jnp.dot(x_ref[...], w1_ref[...], preferred_element_type=jnp.float32) * dis
    h1s_ref[0] = res[:, :HS]
    h1s_ref[1] = res[:, HS:]


def _tc_mid_body(agg_ref, dis_ref, b1_ref, w2a_ref, w2b_ref, h2s_ref):
    dis = dis_ref[...]
    yl = agg_ref[0, 0] + agg_ref[1, 0]
    yr = agg_ref[0, 1] + agg_ref[1, 1]
    rl = jnp.maximum(dis * yl + b1_ref[:, :HS], 0.0)
    rr = jnp.maximum(dis * yr + b1_ref[:, HS:], 0.0)
    h2 = (jnp.dot(rl, w2a_ref[...], preferred_element_type=jnp.float32)
          + jnp.dot(rr, w2b_ref[...], preferred_element_type=jnp.float32))
    h2s_ref[...] = h2 * dis


def _tc_post_body(agg_ref, dis_ref, b2_ref, out_ref):
    y2 = agg_ref[0, 0] + agg_ref[1, 0]
    out_ref[...] = dis_ref[...] * y2[:, : out_ref.shape[1]] + b2_ref[...]


def kernel(x, edge_index, W1, b1, W2, b2):
    src32 = edge_index[0].astype(jnp.int32)
    dst32 = edge_index[1].astype(jnp.int32)
    pad = EP - E
    src2d = jnp.concatenate([src32, jnp.zeros((pad,), jnp.int32)]).reshape(EP // CHUNK, CHUNK)
    pad_dst = N + jnp.arange(pad, dtype=jnp.int32) % (NP - N)
    dst2d = jnp.concatenate([dst32, pad_dst]).reshape(EP // CHUNK, CHUNK)
    zeros1 = jnp.zeros((NP, CP), jnp.float32)
    ones1 = jnp.ones((CHUNK, CP), jnp.float32)
    zeros32 = jnp.zeros((NP, HS), jnp.float32)
    zeros16 = jnp.zeros((NP, CP), jnp.float32)
    W2p = jnp.concatenate([W2, jnp.zeros((H, CP - W2.shape[1]), jnp.float32)], axis=1)
    b1r = b1.reshape(1, H)
    b2r = b2.reshape(1, -1)
    nb = N // BN

    degp = _sc_degree(dst2d, zeros1, ones1)

    h1s, dis = pl.pallas_call(
        _tc_pre_body,
        grid=(nb,),
        in_specs=[
            pl.BlockSpec((BN, F), lambda i: (i, 0)),
            pl.BlockSpec((F, H), lambda i: (0, 0)),
            pl.BlockSpec((NC, BN, CP), lambda i: (0, i, 0)),
        ],
        out_specs=[
            pl.BlockSpec((2, BN, HS), lambda i: (0, i, 0)),
            pl.BlockSpec((BN, 1), lambda i: (i, 0)),
        ],
        out_shape=[
            jax.ShapeDtypeStruct((2, NP, HS), jnp.float32),
            jax.ShapeDtypeStruct((N, 1), jnp.float32),
        ],
    )(x, W1, degp)

    agg1 = _sc_aggregate(2, HS, h1s, src2d, dst2d, zeros32)

    h2s = pl.pallas_call(
        _tc_mid_body,
        grid=(nb,),
        in_specs=[
            pl.BlockSpec((NC, 2, BN, HS), lambda i: (0, 0, i, 0)),
            pl.BlockSpec((BN, 1), lambda i: (i, 0)),
            pl.BlockSpec((1, H), lambda i: (0, 0)),
            pl.BlockSpec((HS, CP), lambda i: (0, 0)),
            pl.BlockSpec((HS, CP), lambda i: (0, 0)),
        ],
        out_specs=pl.BlockSpec((BN, CP), lambda i: (i, 0)),
        out_shape=jax.ShapeDtypeStruct((NP, CP), jnp.float32),
    )(agg1, dis, b1r, W2p[:HS], W2p[HS:])

    agg2 = _sc_aggregate(1, CP, h2s.reshape(1, NP, CP), src2d, dst2d, zeros16)

    out = pl.pallas_call(
        _tc_post_body,
        grid=(nb,),
        in_specs=[
            pl.BlockSpec((NC, 1, BN, CP), lambda i: (0, 0, i, 0)),
            pl.BlockSpec((BN, 1), lambda i: (i, 0)),
            pl.BlockSpec((1, b2.shape[0]), lambda i: (0, 0)),
        ],
        out_specs=pl.BlockSpec((BN, b2.shape[0]), lambda i: (i, 0)),
        out_shape=jax.ShapeDtypeStruct((N, b2.shape[0]), jnp.float32),
    )(agg2, dis, b2r)

    return out
